# baseline (speedup 1.0000x reference)
import functools

import jax
import jax.numpy as jnp
from jax import lax
from jax.experimental import pallas as pl
from jax.experimental.pallas import tpu as pltpu
from jax.experimental.pallas import tpu_sc as plsc

N = 100000
D = 128
K = 50000
M = 131072
ROWS = M // 128

_info = plsc.get_sparse_core_info()
NC, NS = _info.num_cores, _info.num_subcores
NW = NC * NS
PER_W = 1568
CHUNK = 128
NCHUNK = 13


def _loss_body(s_ref, out_ref):
    s = s_ref[...]
    row = jax.lax.broadcasted_iota(jnp.int32, (ROWS, 128), 0)
    lane = jax.lax.broadcasted_iota(jnp.int32, (ROWS, 128), 1)
    g = row * 128 + lane
    eps = 1e-08
    top = jnp.where(g < K, jnp.log(s + eps), 0.0)
    rest = jnp.where((g >= K) & (g < N), jnp.log(1.0 - s + eps), 0.0)
    out_ref[0, 0] = -(jnp.sum(top) + jnp.sum(rest)) / N


def _pool_loss(s_sorted_padded):
    return pl.pallas_call(
        _loss_body,
        out_shape=jax.ShapeDtypeStruct((1, 1), jnp.float32),
        out_specs=pl.BlockSpec(memory_space=pltpu.SMEM),
    )(s_sorted_padded.reshape(ROWS, 128))[0, 0]


def _gather_body(x_hbm, idx_hbm, s_hbm, out_hbm, idx_v, rows_v, s_v, sem):
    wid = lax.axis_index("s") * NC + lax.axis_index("c")
    base = wid * PER_W
    end = jnp.minimum(base + PER_W, K)

    def scale_row(r, _):
        sv = s_v[r, :]
        for j in range(D // 16):
            rows_v[r, pl.ds(j * 16, 16)] = rows_v[r, pl.ds(j * 16, 16)] * sv
        return _

    for c in range(NCHUNK):
        start0 = base + c * CHUNK

        @pl.when(start0 < end)
        def _():
            start = jnp.minimum(start0, end - CHUNK)
            pltpu.sync_copy(idx_hbm.at[pl.ds(start, CHUNK)], idx_v)
            pltpu.async_copy(x_hbm.at[idx_v], rows_v, sem).wait()
            pltpu.sync_copy(s_hbm.at[pl.ds(start, CHUNK)], s_v)
            lax.fori_loop(0, CHUNK, scale_row, 0)
            pltpu.sync_copy(rows_v, out_hbm.at[pl.ds(start, CHUNK)])


_gather_scale = functools.partial(
    pl.kernel,
    out_type=jax.ShapeDtypeStruct((K, D), jnp.float32),
    mesh=plsc.VectorSubcoreMesh(core_axis_name="c", subcore_axis_name="s"),
    scratch_types=[
        pltpu.VMEM((CHUNK,), jnp.int32),
        pltpu.VMEM((CHUNK, D), jnp.float32),
        pltpu.VMEM((CHUNK, 16), jnp.float32),
        pltpu.SemaphoreType.DMA,
    ],
)(_gather_body)


def kernel(lw_matrix_hidden_state_last, trainable_vector_pooling):
    x = lw_matrix_hidden_state_last
    v = trainable_vector_pooling
    norm2 = jnp.linalg.norm(v)
    scores = x @ (v / (norm2 + 1e-08))
    scores = (scores - scores.mean()) / (scores.std() + 1e-08)
    sig_scores = jax.nn.sigmoid(scores)
    s = sig_scores.squeeze(-1)
    _, indices = jax.lax.top_k(s, K)
    s_sorted = -jnp.sort(-s)
    s_exp = jnp.broadcast_to(s_sorted[:K, None], (K, 16))
    new_x = _gather_scale(x, indices, s_exp)
    s_pad = jnp.concatenate([s_sorted, jnp.full((M - N,), 0.5, jnp.float32)])
    pool_loss = _pool_loss(s_pad)
    return (new_x, pool_loss)


# speedup vs baseline: 2.2388x; 2.2388x over previous
import functools

import jax
import jax.numpy as jnp
from jax import lax
from jax.experimental import pallas as pl
from jax.experimental.pallas import tpu as pltpu
from jax.experimental.pallas import tpu_sc as plsc

N = 100000
D = 128
K = 50000

_info = plsc.get_sparse_core_info()
NC, NS = _info.num_cores, _info.num_subcores
NW = NC * NS

NT = 16
TOT = 100352
PT = TOT // NT
NV = PT // 16
NCH = PT // 128
NB = 1024
KBIAS = 1 << 30

PER_W = 1568
CHUNK = 128
NCHUNK = 13

LROWS = TOT // 128


def _loss_body(s_ref, out_ref):
    s = s_ref[...]
    row = jax.lax.broadcasted_iota(jnp.int32, (LROWS, 128), 0)
    lane = jax.lax.broadcasted_iota(jnp.int32, (LROWS, 128), 1)
    g = row * 128 + lane
    eps = 1e-08
    top = jnp.where(g < K, jnp.log(s + eps), 0.0)
    rest = jnp.where((g >= K) & (g < N), jnp.log(1.0 - s + eps), 0.0)
    out_ref[0, 0] = -(jnp.sum(top) + jnp.sum(rest)) / N


def _pool_loss(s_sorted_padded):
    return pl.pallas_call(
        _loss_body,
        out_shape=jax.ShapeDtypeStruct((1, 1), jnp.float32),
        out_specs=pl.BlockSpec(memory_space=pltpu.SMEM),
    )(s_sorted_padded.reshape(LROWS, 128))[0, 0]


def _sort_body(k_hbm, kout_hbm, iout_hbm,
               keys_v, vals_v, pk_v, pv_v, pos_v,
               hist_v, hall_v, tot_v, prio_v, base_v,
               ka, va, kb, vb, hsh, sem):
    core = lax.axis_index("c")
    tid = lax.axis_index("s")
    lane = jnp.arange(16, dtype=jnp.int32)
    lprev = jnp.maximum(lane - 1, 0)
    lnext = jnp.minimum(lane + 1, 15)
    pib = "promise_in_bounds"

    @pl.when(core == 0)
    def _sort():
        tbase = tid * PT

        def hist_groups(sd):
            prevd = sd.at[lprev].get(mode=pib)
            nxtd = sd.at[lnext].get(mode=pib)
            chg = (lane == 0) | (prevd != sd)
            iend = (lane == 15) | (nxtd != sd)
            startp = plsc.cummax(jnp.where(chg, lane, 0))
            rank = lane - startp
            return rank, iend

        for p in range(3):
            shift = 10 * p
            ksrc, vsrc = (ka, va) if p == 1 else (kb, vb)
            kdst, vdst = (kb, vb) if p == 1 else (ka, va)

            for i in range(NB // 16):
                hist_v[pl.ds(i * 16, 16)] = jnp.zeros((16,), jnp.int32)
            if p == 0:
                pltpu.sync_copy(k_hbm.at[pl.ds(tbase, PT)], keys_v)
            else:
                pltpu.sync_copy(ksrc.at[pl.ds(tbase, PT)], keys_v)
                pltpu.sync_copy(vsrc.at[pl.ds(tbase, PT)], vals_v)

            def histo(m, _):
                k16 = keys_v[pl.ds(m * 16, 16)]
                d16 = (k16 >> shift) & (NB - 1)
                sd = lax.sort(d16)
                rank, iend = hist_groups(sd)
                plsc.addupdate_scatter(hist_v, [sd], rank + 1, mask=iend)
                return _

            lax.fori_loop(0, NV, histo, 0)

            pltpu.sync_copy(hist_v, hsh.at[tid])
            plsc.subcore_barrier()
            pltpu.sync_copy(hsh, hall_v)

            def sums(b, _):
                tot = jnp.zeros((16,), jnp.int32)
                pri = jnp.zeros((16,), jnp.int32)
                for t in range(NT):
                    h = hall_v[t, pl.ds(b * 16, 16)]
                    tot = tot + h
                    pri = pri + jnp.where(t < tid, h, 0)
                tot_v[pl.ds(b * 16, 16)] = tot
                prio_v[pl.ds(b * 16, 16)] = pri
                return _

            lax.fori_loop(0, NB // 16, sums, 0)

            def scan(b, carry):
                t16 = tot_v[pl.ds(b * 16, 16)]
                c16 = plsc.cumsum(t16)
                base_v[pl.ds(b * 16, 16)] = (
                    carry + (c16 - t16) + prio_v[pl.ds(b * 16, 16)])
                return carry + jnp.sum(t16)

            lax.fori_loop(0, NB // 16, scan, jnp.int32(0))

            def permute(m, car):
                k16 = keys_v[pl.ds(m * 16, 16)]
                if p == 0:
                    v16 = tbase + m * 16 + lane
                else:
                    v16 = vals_v[pl.ds(m * 16, 16)]
                d16 = (k16 >> shift) & (NB - 1)
                skey = d16 * 16 + lane
                _sk0, pk = plsc.sort_key_val(skey, k16)
                sk, pv = plsc.sort_key_val(skey, v16)
                sd = sk >> 4
                rank, iend = hist_groups(sd)
                b16 = plsc.load_gather(base_v, [sd])
                plsc.addupdate_scatter(base_v, [sd], rank + 1, mask=iend)
                pos = b16 + rank
                pk_v[pl.ds(m * 16, 16)] = pk
                pv_v[pl.ds(m * 16, 16)] = pv
                pos_v[m >> 3, pl.ds((m & 7) * 16, 16)] = pos
                return car

            lax.fori_loop(0, NV, permute, 0)

            copies = []
            for j in range(NCH):
                copies.append(pltpu.async_copy(
                    pk_v.at[pl.ds(j * 128, 128)], kdst.at[pos_v.at[j]], sem))
                copies.append(pltpu.async_copy(
                    pv_v.at[pl.ds(j * 128, 128)], vdst.at[pos_v.at[j]], sem))
            for c in copies:
                c.wait()
            plsc.subcore_barrier()

        pltpu.sync_copy(ka.at[pl.ds(tbase, PT)], keys_v)
        pltpu.sync_copy(keys_v, kout_hbm.at[pl.ds(tbase, PT)])
        pltpu.sync_copy(va.at[pl.ds(tbase, PT)], vals_v)
        pltpu.sync_copy(vals_v, iout_hbm.at[pl.ds(tbase, PT)])


_radix_sort = functools.partial(
    pl.kernel,
    out_type=(jax.ShapeDtypeStruct((TOT,), jnp.int32),
              jax.ShapeDtypeStruct((TOT,), jnp.int32)),
    compiler_params=pltpu.CompilerParams(needs_layout_passes=False),
    mesh=plsc.VectorSubcoreMesh(core_axis_name="c", subcore_axis_name="s"),
    scratch_types=[
        pltpu.VMEM((PT,), jnp.int32),
        pltpu.VMEM((PT,), jnp.int32),
        pltpu.VMEM((PT,), jnp.int32),
        pltpu.VMEM((PT,), jnp.int32),
        pltpu.VMEM((NCH, 128), jnp.int32),
        pltpu.VMEM((NB,), jnp.int32),
        pltpu.VMEM((NT, NB), jnp.int32),
        pltpu.VMEM((NB,), jnp.int32),
        pltpu.VMEM((NB,), jnp.int32),
        pltpu.VMEM((NB,), jnp.int32),
        pltpu.VMEM_SHARED((TOT,), jnp.int32),
        pltpu.VMEM_SHARED((TOT,), jnp.int32),
        pltpu.VMEM_SHARED((TOT,), jnp.int32),
        pltpu.VMEM_SHARED((TOT,), jnp.int32),
        pltpu.VMEM_SHARED((NT, NB), jnp.int32),
        pltpu.SemaphoreType.DMA,
    ],
)(_sort_body)


def _gather_body(x_hbm, idx_hbm, s_hbm, out_hbm, idx_v, rows_v, s_v, sem):
    wid = lax.axis_index("s") * NC + lax.axis_index("c")
    base = wid * PER_W
    end = jnp.minimum(base + PER_W, K)

    def scale_row(r, _):
        sv = s_v[r, :]
        for j in range(D // 16):
            rows_v[r, pl.ds(j * 16, 16)] = rows_v[r, pl.ds(j * 16, 16)] * sv
        return _

    for c in range(NCHUNK):
        start0 = base + c * CHUNK

        @pl.when(start0 < end)
        def _():
            start = jnp.minimum(start0, end - CHUNK)
            pltpu.sync_copy(idx_hbm.at[pl.ds(start, CHUNK)], idx_v)
            pltpu.async_copy(x_hbm.at[idx_v], rows_v, sem).wait()
            pltpu.sync_copy(s_hbm.at[pl.ds(start, CHUNK)], s_v)
            lax.fori_loop(0, CHUNK, scale_row, 0)
            pltpu.sync_copy(rows_v, out_hbm.at[pl.ds(start, CHUNK)])


_gather_scale = functools.partial(
    pl.kernel,
    out_type=jax.ShapeDtypeStruct((K, D), jnp.float32),
    compiler_params=pltpu.CompilerParams(needs_layout_passes=False),
    mesh=plsc.VectorSubcoreMesh(core_axis_name="c", subcore_axis_name="s"),
    scratch_types=[
        pltpu.VMEM((CHUNK,), jnp.int32),
        pltpu.VMEM((CHUNK, D), jnp.float32),
        pltpu.VMEM((CHUNK, 16), jnp.float32),
        pltpu.SemaphoreType.DMA,
    ],
)(_gather_body)


def kernel(lw_matrix_hidden_state_last, trainable_vector_pooling):
    x = lw_matrix_hidden_state_last
    v = trainable_vector_pooling
    norm2 = jnp.linalg.norm(v)
    scores = x @ (v / (norm2 + 1e-08))
    scores = (scores - scores.mean()) / (scores.std() + 1e-08)
    sig_scores = jax.nn.sigmoid(scores)
    s = sig_scores.squeeze(-1)
    bits = jax.lax.bitcast_convert_type(s, jnp.int32)
    keys_in = jnp.concatenate(
        [KBIAS - bits, jnp.full((TOT - N,), KBIAS - 1, jnp.int32)])
    kout, idx_sorted = _radix_sort(keys_in)
    s_sorted = jax.lax.bitcast_convert_type(KBIAS - kout, jnp.float32)
    s_exp = jnp.broadcast_to(s_sorted[:K, None], (K, 16))
    new_x = _gather_scale(x, idx_sorted[:K], s_exp)
    pool_loss = _pool_loss(s_sorted)
    return (new_x, pool_loss)


# speedup vs baseline: 2.4318x; 1.0862x over previous
import functools

import jax
import jax.numpy as jnp
from jax import lax
from jax.experimental import pallas as pl
from jax.experimental.pallas import tpu as pltpu
from jax.experimental.pallas import tpu_sc as plsc

N = 100000
D = 128
K = 50000

_info = plsc.get_sparse_core_info()
NC, NS = _info.num_cores, _info.num_subcores
NW = NC * NS

NT = 16
TOT = 100352
PT = TOT // NT
NV = PT // 16
NCH = PT // 128
NB = 1024
KBIAS = 1 << 30

PER_W = 1568
CHUNK = 128
NCHUNK = 13

LROWS = TOT // 128


def _loss_body(s_ref, out_ref):
    s = s_ref[...]
    row = jax.lax.broadcasted_iota(jnp.int32, (LROWS, 128), 0)
    lane = jax.lax.broadcasted_iota(jnp.int32, (LROWS, 128), 1)
    g = row * 128 + lane
    eps = 1e-08
    top = jnp.where(g < K, jnp.log(s + eps), 0.0)
    rest = jnp.where((g >= K) & (g < N), jnp.log(1.0 - s + eps), 0.0)
    out_ref[0, 0] = -(jnp.sum(top) + jnp.sum(rest)) / N


def _pool_loss(s_sorted_padded):
    return pl.pallas_call(
        _loss_body,
        out_shape=jax.ShapeDtypeStruct((1, 1), jnp.float32),
        out_specs=pl.BlockSpec(memory_space=pltpu.SMEM),
    )(s_sorted_padded.reshape(LROWS, 128))[0, 0]


def _sort_body(k_hbm, kout_hbm, iout_hbm,
               keys_v, vals_v, pk_v, pv_v, pos_v,
               hist_v, hall_v, tot_v, prio_v, base_v,
               ka, va, kb, vb, hsh, sem):
    core = lax.axis_index("c")
    tid = lax.axis_index("s")
    lane = jnp.arange(16, dtype=jnp.int32)
    lprev = jnp.maximum(lane - 1, 0)
    lnext = jnp.minimum(lane + 1, 15)
    pib = "promise_in_bounds"

    @pl.when(core == 0)
    def _sort():
        tbase = tid * PT

        def hist_groups(sd):
            prevd = sd.at[lprev].get(mode=pib)
            nxtd = sd.at[lnext].get(mode=pib)
            chg = (lane == 0) | (prevd != sd)
            iend = (lane == 15) | (nxtd != sd)
            startp = plsc.cummax(jnp.where(chg, lane, 0))
            rank = lane - startp
            return rank, iend

        for p in range(3):
            shift = 10 * p
            ksrc, vsrc = (ka, va) if p == 1 else (kb, vb)
            kdst, vdst = (kb, vb) if p == 1 else (ka, va)

            for i in range(NB // 16):
                hist_v[pl.ds(i * 16, 16)] = jnp.zeros((16,), jnp.int32)
            if p == 0:
                pltpu.sync_copy(k_hbm.at[pl.ds(tbase, PT)], keys_v)
            else:
                pltpu.sync_copy(ksrc.at[pl.ds(tbase, PT)], keys_v)
                pltpu.sync_copy(vsrc.at[pl.ds(tbase, PT)], vals_v)

            def histo(m, _):
                k16 = keys_v[pl.ds(m * 16, 16)]
                d16 = (k16 >> shift) & (NB - 1)
                sd = lax.sort(d16)
                rank, iend = hist_groups(sd)
                plsc.addupdate_scatter(hist_v, [sd], rank + 1, mask=iend)
                return _

            lax.fori_loop(0, NV, histo, 0)

            pltpu.sync_copy(hist_v, hsh.at[tid])
            plsc.subcore_barrier()
            pltpu.sync_copy(hsh, hall_v)

            def sums(b, _):
                tot = jnp.zeros((16,), jnp.int32)
                pri = jnp.zeros((16,), jnp.int32)
                for t in range(NT):
                    h = hall_v[t, pl.ds(b * 16, 16)]
                    tot = tot + h
                    pri = pri + jnp.where(t < tid, h, 0)
                tot_v[pl.ds(b * 16, 16)] = tot
                prio_v[pl.ds(b * 16, 16)] = pri
                return _

            lax.fori_loop(0, NB // 16, sums, 0)

            def scan(b, carry):
                t16 = tot_v[pl.ds(b * 16, 16)]
                c16 = plsc.cumsum(t16)
                base_v[pl.ds(b * 16, 16)] = (
                    carry + (c16 - t16) + prio_v[pl.ds(b * 16, 16)])
                return carry + jnp.sum(t16)

            lax.fori_loop(0, NB // 16, scan, jnp.int32(0))

            def permute(m, car):
                k16 = keys_v[pl.ds(m * 16, 16)]
                if p == 0:
                    v16 = tbase + m * 16 + lane
                else:
                    v16 = vals_v[pl.ds(m * 16, 16)]
                d16 = (k16 >> shift) & (NB - 1)
                skey = d16 * 16 + lane
                _sk0, pk = plsc.sort_key_val(skey, k16)
                sk, pv = plsc.sort_key_val(skey, v16)
                sd = sk >> 4
                rank, iend = hist_groups(sd)
                b16 = plsc.load_gather(base_v, [sd])
                plsc.addupdate_scatter(base_v, [sd], rank + 1, mask=iend)
                pos = b16 + rank
                pk_v[pl.ds(m * 16, 16)] = pk
                pv_v[pl.ds(m * 16, 16)] = pv
                pos_v[m >> 3, pl.ds((m & 7) * 16, 16)] = pos
                return car

            lax.fori_loop(0, NV, permute, 0)

            copies = []
            for j in range(NCH):
                copies.append(pltpu.async_copy(
                    pk_v.at[pl.ds(j * 128, 128)], kdst.at[pos_v.at[j]], sem))
                copies.append(pltpu.async_copy(
                    pv_v.at[pl.ds(j * 128, 128)], vdst.at[pos_v.at[j]], sem))
            for c in copies:
                c.wait()
            plsc.subcore_barrier()

        pltpu.sync_copy(ka.at[pl.ds(tbase, PT)], keys_v)
        pltpu.sync_copy(keys_v, kout_hbm.at[pl.ds(tbase, PT)])
        pltpu.sync_copy(va.at[pl.ds(tbase, PT)], vals_v)
        pltpu.sync_copy(vals_v, iout_hbm.at[pl.ds(tbase, PT)])


_radix_sort = functools.partial(
    pl.kernel,
    out_type=(jax.ShapeDtypeStruct((TOT,), jnp.int32),
              jax.ShapeDtypeStruct((TOT,), jnp.int32)),
    compiler_params=pltpu.CompilerParams(needs_layout_passes=False),
    mesh=plsc.VectorSubcoreMesh(core_axis_name="c", subcore_axis_name="s"),
    scratch_types=[
        pltpu.VMEM((PT,), jnp.int32),
        pltpu.VMEM((PT,), jnp.int32),
        pltpu.VMEM((PT,), jnp.int32),
        pltpu.VMEM((PT,), jnp.int32),
        pltpu.VMEM((NCH, 128), jnp.int32),
        pltpu.VMEM((NB,), jnp.int32),
        pltpu.VMEM((NT, NB), jnp.int32),
        pltpu.VMEM((NB,), jnp.int32),
        pltpu.VMEM((NB,), jnp.int32),
        pltpu.VMEM((NB,), jnp.int32),
        pltpu.VMEM_SHARED((TOT,), jnp.int32),
        pltpu.VMEM_SHARED((TOT,), jnp.int32),
        pltpu.VMEM_SHARED((TOT,), jnp.int32),
        pltpu.VMEM_SHARED((TOT,), jnp.int32),
        pltpu.VMEM_SHARED((NT, NB), jnp.int32),
        pltpu.SemaphoreType.DMA,
    ],
)(_sort_body)


def _gather_body(x_hbm, idx_hbm, s_hbm, out_hbm,
                 idx0, idx1, rows0, rows1, s0, s1, sem0, sem1):
    wid = lax.axis_index("s") * NC + lax.axis_index("c")
    base = wid * PER_W
    end = jnp.minimum(base + PER_W, K)
    idx_b = (idx0, idx1)
    rows_b = (rows0, rows1)
    s_b = (s0, s1)
    sem_b = (sem0, sem1)

    def chunk_start(c):
        return jnp.minimum(base + c * CHUNK, end - CHUNK)

    def issue(c):
        b = c & 1
        start = chunk_start(c)
        pltpu.sync_copy(idx_hbm.at[pl.ds(start, CHUNK)], idx_b[b])
        pltpu.sync_copy(s_hbm.at[pl.ds(start, CHUNK)], s_b[b])
        return pltpu.async_copy(x_hbm.at[idx_b[b]], rows_b[b], sem_b[b])

    def make_scale(rows_v, s_v):
        def scale_row(r, car):
            sv = s_v[r, :]
            for j in range(D // 16):
                rows_v[r, pl.ds(j * 16, 16)] = (
                    rows_v[r, pl.ds(j * 16, 16)] * sv)
            return car
        return scale_row

    pend = [None, None]
    pend[0] = issue(0)
    for c in range(NCHUNK):
        b = c & 1
        if c + 1 < NCHUNK:
            pend[1 - b] = issue(c + 1)
        pend[b].wait()
        lax.fori_loop(0, CHUNK, make_scale(rows_b[b], s_b[b]), 0)
        pltpu.sync_copy(rows_b[b], out_hbm.at[pl.ds(chunk_start(c), CHUNK)])


_gather_scale = functools.partial(
    pl.kernel,
    out_type=jax.ShapeDtypeStruct((K, D), jnp.float32),
    compiler_params=pltpu.CompilerParams(needs_layout_passes=False),
    mesh=plsc.VectorSubcoreMesh(core_axis_name="c", subcore_axis_name="s"),
    scratch_types=[
        pltpu.VMEM((CHUNK,), jnp.int32),
        pltpu.VMEM((CHUNK,), jnp.int32),
        pltpu.VMEM((CHUNK, D), jnp.float32),
        pltpu.VMEM((CHUNK, D), jnp.float32),
        pltpu.VMEM((CHUNK, 16), jnp.float32),
        pltpu.VMEM((CHUNK, 16), jnp.float32),
        pltpu.SemaphoreType.DMA,
        pltpu.SemaphoreType.DMA,
    ],
)(_gather_body)


def kernel(lw_matrix_hidden_state_last, trainable_vector_pooling):
    x = lw_matrix_hidden_state_last
    v = trainable_vector_pooling
    norm2 = jnp.linalg.norm(v)
    scores = x @ (v / (norm2 + 1e-08))
    scores = (scores - scores.mean()) / (scores.std() + 1e-08)
    sig_scores = jax.nn.sigmoid(scores)
    s = sig_scores.squeeze(-1)
    bits = jax.lax.bitcast_convert_type(s, jnp.int32)
    keys_in = jnp.concatenate(
        [KBIAS - bits, jnp.full((TOT - N,), KBIAS - 1, jnp.int32)])
    kout, idx_sorted = _radix_sort(keys_in)
    s_sorted = jax.lax.bitcast_convert_type(KBIAS - kout, jnp.float32)
    s_exp = jnp.broadcast_to(s_sorted[:K, None], (K, 16))
    new_x = _gather_scale(x, idx_sorted[:K], s_exp)
    pool_loss = _pool_loss(s_sorted)
    return (new_x, pool_loss)


# speedup vs baseline: 3.0864x; 1.2692x over previous
import functools

import jax
import jax.numpy as jnp
from jax import lax
from jax.experimental import pallas as pl
from jax.experimental.pallas import tpu as pltpu
from jax.experimental.pallas import tpu_sc as plsc

N = 100000
D = 128
K = 50000

_info = plsc.get_sparse_core_info()
NC, NS = _info.num_cores, _info.num_subcores
NW = NC * NS

NT = 16
TOT = 100352
PT = TOT // NT
NV = PT // 16
NCH = PT // 128
NB = 1024
KBIAS = 1 << 30

PER_W = 1568
CHUNK = 128
NCHUNK = 13

LROWS = TOT // 128


def _loss_body(s_ref, out_ref):
    s = s_ref[...]
    row = jax.lax.broadcasted_iota(jnp.int32, (LROWS, 128), 0)
    lane = jax.lax.broadcasted_iota(jnp.int32, (LROWS, 128), 1)
    g = row * 128 + lane
    eps = 1e-08
    top = jnp.where(g < K, jnp.log(s + eps), 0.0)
    rest = jnp.where((g >= K) & (g < N), jnp.log(1.0 - s + eps), 0.0)
    out_ref[0, 0] = -(jnp.sum(top) + jnp.sum(rest)) / N


def _pool_loss(s_sorted_padded):
    return pl.pallas_call(
        _loss_body,
        out_shape=jax.ShapeDtypeStruct((1, 1), jnp.float32),
        out_specs=pl.BlockSpec(memory_space=pltpu.SMEM),
    )(s_sorted_padded.reshape(LROWS, 128))[0, 0]


def _sort_body(k_hbm, kout_hbm, iout_hbm,
               keys_v, vals_v, pk_v, pv_v, pos_v,
               hist_v, hall_v, tot_v, prio_v, base_v,
               ka, va, kb, vb, hsh, sem):
    core = lax.axis_index("c")
    tid = lax.axis_index("s")
    lane = jnp.arange(16, dtype=jnp.int32)
    lprev = jnp.maximum(lane - 1, 0)
    lnext = jnp.minimum(lane + 1, 15)
    pib = "promise_in_bounds"

    @pl.when(core == 0)
    def _sort():
        tbase = tid * PT

        def hist_groups(sd):
            prevd = sd.at[lprev].get(mode=pib)
            nxtd = sd.at[lnext].get(mode=pib)
            chg = (lane == 0) | (prevd != sd)
            iend = (lane == 15) | (nxtd != sd)
            startp = plsc.cummax(jnp.where(chg, lane, 0))
            rank = lane - startp
            return rank, iend

        for p in range(3):
            shift = 10 * p
            ksrc, vsrc = (ka, va) if p == 1 else (kb, vb)
            kdst, vdst = (kb, vb) if p == 1 else (ka, va)

            for i in range(NB // 16):
                hist_v[pl.ds(i * 16, 16)] = jnp.zeros((16,), jnp.int32)
            if p == 0:
                pltpu.sync_copy(k_hbm.at[pl.ds(tbase, PT)], keys_v)
            else:
                pltpu.sync_copy(ksrc.at[pl.ds(tbase, PT)], keys_v)
                pltpu.sync_copy(vsrc.at[pl.ds(tbase, PT)], vals_v)

            def histo(m, _):
                k16 = keys_v[pl.ds(m * 16, 16)]
                d16 = (k16 >> shift) & (NB - 1)
                plsc.addupdate_scatter(hist_v, [d16], jnp.ones((16,), jnp.int32))
                return _

            lax.fori_loop(0, NV, histo, 0)

            pltpu.sync_copy(hist_v, hsh.at[tid])
            plsc.subcore_barrier()
            pltpu.sync_copy(hsh, hall_v)

            def sums(b, _):
                tot = jnp.zeros((16,), jnp.int32)
                pri = jnp.zeros((16,), jnp.int32)
                for t in range(NT):
                    h = hall_v[t, pl.ds(b * 16, 16)]
                    tot = tot + h
                    pri = pri + jnp.where(t < tid, h, 0)
                tot_v[pl.ds(b * 16, 16)] = tot
                prio_v[pl.ds(b * 16, 16)] = pri
                return _

            lax.fori_loop(0, NB // 16, sums, 0)

            def scan(b, carry):
                t16 = tot_v[pl.ds(b * 16, 16)]
                c16 = plsc.cumsum(t16)
                base_v[pl.ds(b * 16, 16)] = (
                    carry + (c16 - t16) + prio_v[pl.ds(b * 16, 16)])
                return carry + jnp.sum(t16)

            lax.fori_loop(0, NB // 16, scan, jnp.int32(0))

            def permute(m, car):
                k16 = keys_v[pl.ds(m * 16, 16)]
                if p == 0:
                    v16 = tbase + m * 16 + lane
                else:
                    v16 = vals_v[pl.ds(m * 16, 16)]
                d16 = (k16 >> shift) & (NB - 1)
                skey = d16 * 16 + lane
                _sk0, pk = plsc.sort_key_val(skey, k16)
                sk, pv = plsc.sort_key_val(skey, v16)
                sd = sk >> 4
                rank, iend = hist_groups(sd)
                b16 = plsc.load_gather(base_v, [sd])
                plsc.addupdate_scatter(base_v, [sd], rank + 1, mask=iend)
                pos = b16 + rank
                pk_v[pl.ds(m * 16, 16)] = pk
                pv_v[pl.ds(m * 16, 16)] = pv
                pos_v[m >> 3, pl.ds((m & 7) * 16, 16)] = pos
                return car

            lax.fori_loop(0, NV, permute, 0)

            copies = []
            for j in range(NCH):
                copies.append(pltpu.async_copy(
                    pk_v.at[pl.ds(j * 128, 128)], kdst.at[pos_v.at[j]], sem))
                copies.append(pltpu.async_copy(
                    pv_v.at[pl.ds(j * 128, 128)], vdst.at[pos_v.at[j]], sem))
            for c in copies:
                c.wait()
            plsc.subcore_barrier()

        pltpu.sync_copy(ka.at[pl.ds(tbase, PT)], keys_v)
        pltpu.sync_copy(keys_v, kout_hbm.at[pl.ds(tbase, PT)])
        pltpu.sync_copy(va.at[pl.ds(tbase, PT)], vals_v)
        pltpu.sync_copy(vals_v, iout_hbm.at[pl.ds(tbase, PT)])


_radix_sort = functools.partial(
    pl.kernel,
    out_type=(jax.ShapeDtypeStruct((TOT,), jnp.int32),
              jax.ShapeDtypeStruct((TOT,), jnp.int32)),
    compiler_params=pltpu.CompilerParams(needs_layout_passes=False),
    mesh=plsc.VectorSubcoreMesh(core_axis_name="c", subcore_axis_name="s"),
    scratch_types=[
        pltpu.VMEM((PT,), jnp.int32),
        pltpu.VMEM((PT,), jnp.int32),
        pltpu.VMEM((PT,), jnp.int32),
        pltpu.VMEM((PT,), jnp.int32),
        pltpu.VMEM((NCH, 128), jnp.int32),
        pltpu.VMEM((NB,), jnp.int32),
        pltpu.VMEM((NT, NB), jnp.int32),
        pltpu.VMEM((NB,), jnp.int32),
        pltpu.VMEM((NB,), jnp.int32),
        pltpu.VMEM((NB,), jnp.int32),
        pltpu.VMEM_SHARED((TOT,), jnp.int32),
        pltpu.VMEM_SHARED((TOT,), jnp.int32),
        pltpu.VMEM_SHARED((TOT,), jnp.int32),
        pltpu.VMEM_SHARED((TOT,), jnp.int32),
        pltpu.VMEM_SHARED((NT, NB), jnp.int32),
        pltpu.SemaphoreType.DMA,
    ],
)(_sort_body)


def _gather_body(x_hbm, idx_hbm, s_hbm, out_hbm,
                 idx0, idx1, rows0, rows1, s0, s1, sem0, sem1):
    wid = lax.axis_index("s") * NC + lax.axis_index("c")
    base = wid * PER_W
    end = jnp.minimum(base + PER_W, K)
    idx_b = (idx0, idx1)
    rows_b = (rows0, rows1)
    s_b = (s0, s1)
    sem_b = (sem0, sem1)

    def chunk_start(c):
        return jnp.minimum(base + c * CHUNK, end - CHUNK)

    def issue(c):
        b = c & 1
        start = chunk_start(c)
        pltpu.sync_copy(idx_hbm.at[pl.ds(start, CHUNK)], idx_b[b])
        pltpu.sync_copy(s_hbm.at[pl.ds(start, CHUNK)], s_b[b])
        return pltpu.async_copy(x_hbm.at[idx_b[b]], rows_b[b], sem_b[b])

    def make_scale(rows_v, s_v):
        def scale_row(r, car):
            sv = plsc.load_gather(s_v, [jnp.broadcast_to(r, (16,))])
            for j in range(D // 16):
                rows_v[r, pl.ds(j * 16, 16)] = (
                    rows_v[r, pl.ds(j * 16, 16)] * sv)
            return car
        return scale_row

    pend = [None, None]
    pend[0] = issue(0)
    for c in range(NCHUNK):
        b = c & 1
        if c + 1 < NCHUNK:
            pend[1 - b] = issue(c + 1)
        pend[b].wait()
        lax.fori_loop(0, CHUNK, make_scale(rows_b[b], s_b[b]), 0)
        pltpu.sync_copy(rows_b[b], out_hbm.at[pl.ds(chunk_start(c), CHUNK)])


_gather_scale = functools.partial(
    pl.kernel,
    out_type=jax.ShapeDtypeStruct((K, D), jnp.float32),
    compiler_params=pltpu.CompilerParams(needs_layout_passes=False),
    mesh=plsc.VectorSubcoreMesh(core_axis_name="c", subcore_axis_name="s"),
    scratch_types=[
        pltpu.VMEM((CHUNK,), jnp.int32),
        pltpu.VMEM((CHUNK,), jnp.int32),
        pltpu.VMEM((CHUNK, D), jnp.float32),
        pltpu.VMEM((CHUNK, D), jnp.float32),
        pltpu.VMEM((CHUNK,), jnp.float32),
        pltpu.VMEM((CHUNK,), jnp.float32),
        pltpu.SemaphoreType.DMA,
        pltpu.SemaphoreType.DMA,
    ],
)(_gather_body)


def kernel(lw_matrix_hidden_state_last, trainable_vector_pooling):
    x = lw_matrix_hidden_state_last
    v = trainable_vector_pooling
    norm2 = jnp.linalg.norm(v)
    scores = x @ (v / (norm2 + 1e-08))
    scores = (scores - scores.mean()) / (scores.std() + 1e-08)
    sig_scores = jax.nn.sigmoid(scores)
    s = sig_scores.squeeze(-1)
    bits = jax.lax.bitcast_convert_type(s, jnp.int32)
    keys_in = jnp.concatenate(
        [KBIAS - bits, jnp.full((TOT - N,), KBIAS - 1, jnp.int32)])
    kout, idx_sorted = _radix_sort(keys_in)
    s_sorted = jax.lax.bitcast_convert_type(KBIAS - kout, jnp.float32)
    new_x = _gather_scale(x, idx_sorted, s_sorted)
    pool_loss = _pool_loss(s_sorted)
    return (new_x, pool_loss)


# speedup vs baseline: 3.3676x; 1.0911x over previous
import functools

import jax
import jax.numpy as jnp
from jax import lax
from jax.experimental import pallas as pl
from jax.experimental.pallas import tpu as pltpu
from jax.experimental.pallas import tpu_sc as plsc

N = 100000
D = 128
K = 50000

_info = plsc.get_sparse_core_info()
NC, NS = _info.num_cores, _info.num_subcores
NW = NC * NS

NT = 16
TOT = 100352
PT = TOT // NT
NV = PT // 16
NCH = PT // 128
NB = 1024
KBIAS = 1 << 30

PER_W = 1568
CHUNK = 128
NCHUNK = 13

LROWS = TOT // 128


def _loss_body(s_ref, out_ref):
    s = s_ref[...]
    row = jax.lax.broadcasted_iota(jnp.int32, (LROWS, 128), 0)
    lane = jax.lax.broadcasted_iota(jnp.int32, (LROWS, 128), 1)
    g = row * 128 + lane
    eps = 1e-08
    top = jnp.where(g < K, jnp.log(s + eps), 0.0)
    rest = jnp.where((g >= K) & (g < N), jnp.log(1.0 - s + eps), 0.0)
    out_ref[0, 0] = -(jnp.sum(top) + jnp.sum(rest)) / N


def _pool_loss(s_sorted_padded):
    return pl.pallas_call(
        _loss_body,
        out_shape=jax.ShapeDtypeStruct((1, 1), jnp.float32),
        out_specs=pl.BlockSpec(memory_space=pltpu.SMEM),
    )(s_sorted_padded.reshape(LROWS, 128))[0, 0]


def _sort_body(k_hbm, kout_hbm, iout_hbm,
               keys_v, vals_v, pk_v, pv_v, pos_v,
               hist_v, hall_v, tot_v, prio_v, base_v,
               ka, va, kb, vb, hsh, sem):
    core = lax.axis_index("c")
    tid = lax.axis_index("s")
    lane = jnp.arange(16, dtype=jnp.int32)

    @pl.when(core == 0)
    def _sort():
        tbase = tid * PT

        for p in range(3):
            shift = 10 * p
            ksrc, vsrc = (ka, va) if p == 1 else (kb, vb)
            kdst, vdst = (kb, vb) if p == 1 else (ka, va)

            for i in range(NB // 16):
                hist_v[pl.ds(i * 16, 16)] = jnp.zeros((16,), jnp.int32)
            if p == 0:
                pltpu.sync_copy(k_hbm.at[pl.ds(tbase, PT)], keys_v)
            else:
                pltpu.sync_copy(ksrc.at[pl.ds(tbase, PT)], keys_v)
                pltpu.sync_copy(vsrc.at[pl.ds(tbase, PT)], vals_v)

            def histo(m, _):
                k16 = keys_v[pl.ds(m * 16, 16)]
                d16 = (k16 >> shift) & (NB - 1)
                plsc.addupdate_scatter(hist_v, [d16], jnp.ones((16,), jnp.int32))
                return _

            lax.fori_loop(0, NV, histo, 0)

            pltpu.sync_copy(hist_v, hsh.at[tid])
            plsc.subcore_barrier()
            pltpu.sync_copy(hsh, hall_v)

            def sums(b, _):
                tot = jnp.zeros((16,), jnp.int32)
                pri = jnp.zeros((16,), jnp.int32)
                for t in range(NT):
                    h = hall_v[t, pl.ds(b * 16, 16)]
                    tot = tot + h
                    pri = pri + jnp.where(t < tid, h, 0)
                tot_v[pl.ds(b * 16, 16)] = tot
                prio_v[pl.ds(b * 16, 16)] = pri
                return _

            lax.fori_loop(0, NB // 16, sums, 0)

            def scan(b, carry):
                t16 = tot_v[pl.ds(b * 16, 16)]
                c16 = plsc.cumsum(t16)
                base_v[pl.ds(b * 16, 16)] = (
                    carry + (c16 - t16) + prio_v[pl.ds(b * 16, 16)])
                return carry + jnp.sum(t16)

            lax.fori_loop(0, NB // 16, scan, jnp.int32(0))

            def permute(m, car):
                k16 = keys_v[pl.ds(m * 16, 16)]
                if p == 0:
                    v16 = tbase + m * 16 + lane
                else:
                    v16 = vals_v[pl.ds(m * 16, 16)]
                d16 = (k16 >> shift) & (NB - 1)
                cnt, _last = plsc.scan_count(d16)
                b16 = plsc.load_gather(base_v, [d16])
                plsc.addupdate_scatter(
                    base_v, [d16], jnp.ones((16,), jnp.int32))
                pos = b16 + (cnt - 1)
                pk_v[pl.ds(m * 16, 16)] = k16
                pv_v[pl.ds(m * 16, 16)] = v16
                pos_v[m >> 3, pl.ds((m & 7) * 16, 16)] = pos
                return car

            lax.fori_loop(0, NV, permute, 0)

            copies = []
            for j in range(NCH):
                copies.append(pltpu.async_copy(
                    pk_v.at[pl.ds(j * 128, 128)], kdst.at[pos_v.at[j]], sem))
                copies.append(pltpu.async_copy(
                    pv_v.at[pl.ds(j * 128, 128)], vdst.at[pos_v.at[j]], sem))
            for c in copies:
                c.wait()
            plsc.subcore_barrier()

        pltpu.sync_copy(ka.at[pl.ds(tbase, PT)], keys_v)
        pltpu.sync_copy(keys_v, kout_hbm.at[pl.ds(tbase, PT)])
        pltpu.sync_copy(va.at[pl.ds(tbase, PT)], vals_v)
        pltpu.sync_copy(vals_v, iout_hbm.at[pl.ds(tbase, PT)])


_radix_sort = functools.partial(
    pl.kernel,
    out_type=(jax.ShapeDtypeStruct((TOT,), jnp.int32),
              jax.ShapeDtypeStruct((TOT,), jnp.int32)),
    compiler_params=pltpu.CompilerParams(needs_layout_passes=False),
    mesh=plsc.VectorSubcoreMesh(core_axis_name="c", subcore_axis_name="s"),
    scratch_types=[
        pltpu.VMEM((PT,), jnp.int32),
        pltpu.VMEM((PT,), jnp.int32),
        pltpu.VMEM((PT,), jnp.int32),
        pltpu.VMEM((PT,), jnp.int32),
        pltpu.VMEM((NCH, 128), jnp.int32),
        pltpu.VMEM((NB,), jnp.int32),
        pltpu.VMEM((NT, NB), jnp.int32),
        pltpu.VMEM((NB,), jnp.int32),
        pltpu.VMEM((NB,), jnp.int32),
        pltpu.VMEM((NB,), jnp.int32),
        pltpu.VMEM_SHARED((TOT,), jnp.int32),
        pltpu.VMEM_SHARED((TOT,), jnp.int32),
        pltpu.VMEM_SHARED((TOT,), jnp.int32),
        pltpu.VMEM_SHARED((TOT,), jnp.int32),
        pltpu.VMEM_SHARED((NT, NB), jnp.int32),
        pltpu.SemaphoreType.DMA,
    ],
)(_sort_body)


def _gather_body(x_hbm, idx_hbm, s_hbm, out_hbm,
                 idx0, idx1, rows0, rows1, s0, s1, sem0, sem1):
    wid = lax.axis_index("s") * NC + lax.axis_index("c")
    base = wid * PER_W
    end = jnp.minimum(base + PER_W, K)
    idx_b = (idx0, idx1)
    rows_b = (rows0, rows1)
    s_b = (s0, s1)
    sem_b = (sem0, sem1)

    def chunk_start(c):
        return jnp.minimum(base + c * CHUNK, end - CHUNK)

    def issue(c):
        b = c & 1
        start = chunk_start(c)
        pltpu.sync_copy(idx_hbm.at[pl.ds(start, CHUNK)], idx_b[b])
        pltpu.sync_copy(s_hbm.at[pl.ds(start, CHUNK)], s_b[b])
        return pltpu.async_copy(x_hbm.at[idx_b[b]], rows_b[b], sem_b[b])

    def make_scale(rows_v, s_v):
        def scale_row(r, car):
            sv = plsc.load_gather(s_v, [jnp.broadcast_to(r, (16,))])
            for j in range(D // 16):
                rows_v[r, pl.ds(j * 16, 16)] = (
                    rows_v[r, pl.ds(j * 16, 16)] * sv)
            return car
        return scale_row

    pend = [None, None]
    pend[0] = issue(0)
    for c in range(NCHUNK):
        b = c & 1
        if c + 1 < NCHUNK:
            pend[1 - b] = issue(c + 1)
        pend[b].wait()
        lax.fori_loop(0, CHUNK, make_scale(rows_b[b], s_b[b]), 0)
        pltpu.sync_copy(rows_b[b], out_hbm.at[pl.ds(chunk_start(c), CHUNK)])


_gather_scale = functools.partial(
    pl.kernel,
    out_type=jax.ShapeDtypeStruct((K, D), jnp.float32),
    compiler_params=pltpu.CompilerParams(needs_layout_passes=False),
    mesh=plsc.VectorSubcoreMesh(core_axis_name="c", subcore_axis_name="s"),
    scratch_types=[
        pltpu.VMEM((CHUNK,), jnp.int32),
        pltpu.VMEM((CHUNK,), jnp.int32),
        pltpu.VMEM((CHUNK, D), jnp.float32),
        pltpu.VMEM((CHUNK, D), jnp.float32),
        pltpu.VMEM((CHUNK,), jnp.float32),
        pltpu.VMEM((CHUNK,), jnp.float32),
        pltpu.SemaphoreType.DMA,
        pltpu.SemaphoreType.DMA,
    ],
)(_gather_body)


def kernel(lw_matrix_hidden_state_last, trainable_vector_pooling):
    x = lw_matrix_hidden_state_last
    v = trainable_vector_pooling
    norm2 = jnp.linalg.norm(v)
    scores = x @ (v / (norm2 + 1e-08))
    scores = (scores - scores.mean()) / (scores.std() + 1e-08)
    sig_scores = jax.nn.sigmoid(scores)
    s = sig_scores.squeeze(-1)
    bits = jax.lax.bitcast_convert_type(s, jnp.int32)
    keys_in = jnp.concatenate(
        [KBIAS - bits, jnp.full((TOT - N,), KBIAS - 1, jnp.int32)])
    kout, idx_sorted = _radix_sort(keys_in)
    s_sorted = jax.lax.bitcast_convert_type(KBIAS - kout, jnp.float32)
    new_x = _gather_scale(x, idx_sorted, s_sorted)
    pool_loss = _pool_loss(s_sorted)
    return (new_x, pool_loss)
